# baseline (speedup 1.0000x reference)
import functools

import jax
import jax.numpy as jnp
from jax import lax
from jax.experimental import pallas as pl
from jax.experimental.pallas import tpu as pltpu
from jax.experimental.pallas import tpu_sc as plsc

L = 16
NC = 2
NS = 16
NW = NC * NS

T = 16384
D = 128
R = 16
V = 1000
LBL = 100000

TPW = T // NW
APT = 64
ABASE_MAX = V - APT

_mesh = plsc.VectorSubcoreMesh(core_axis_name="c", subcore_axis_name="s")


@functools.partial(
    pl.kernel,
    out_type=jax.ShapeDtypeStruct((R, T), jnp.float32),
    mesh=_mesh,
    compiler_params=pltpu.CompilerParams(
        needs_layout_passes=False, use_tc_tiling_on_sc=False),
    scratch_types=[
        pltpu.VMEM((APT, 1, D), jnp.float32),
        pltpu.VMEM((D,), jnp.float32),
        pltpu.VMEM((L,), jnp.float32),
        pltpu.VMEM((TPW,), jnp.int32),
        pltpu.VMEM((TPW,), jnp.int32),
        pltpu.VMEM((R, TPW), jnp.float32),
        pltpu.VMEM((NS * APT,), jnp.float32),
        pltpu.VMEM((APT,), jnp.float32),
        pltpu.VMEM_SHARED((NS * APT,), jnp.float32),
        pltpu.SemaphoreType.DMA,
        pltpu.SemaphoreType.DMA,
    ],
)
def _sc_score(ability_hbm, labels_hbm, w_hbm, bias_hbm, wkr_hbm, tsk_hbm,
              out_hbm, abuf, w_v, bias_v, wkr_v, tsk_v, rows_t, proj_v,
              pbuf, proj_shared, sem, osem):
    cid = lax.axis_index("c")
    sid = lax.axis_index("s")
    wid = sid * NC + cid
    off = wid * TPW

    pltpu.sync_copy(wkr_hbm.at[pl.ds(off, TPW)], wkr_v)
    pltpu.sync_copy(tsk_hbm.at[pl.ds(off, TPW)], tsk_v)
    groups = []
    for g in range(TPW // 128):
        grp = []
        for r in range(R):
            cp = pltpu.make_async_copy(
                labels_hbm.at[pl.ds(r * LBL, LBL)]
                          .at[tsk_v.at[pl.ds(g * 128, 128)]],
                rows_t.at[r, pl.ds(g * 128, 128)], sem)
            cp.start()
            grp.append(cp)
        groups.append(grp)

    pltpu.sync_copy(w_hbm, w_v)
    pltpu.sync_copy(bias_hbm, bias_v)
    base = jnp.minimum(sid * APT, ABASE_MAX)
    pltpu.sync_copy(ability_hbm.at[pl.ds(base, APT)], abuf)
    wch = [w_v[pl.ds(c * L, L)] for c in range(D // L)]

    lane_iota = lax.iota(jnp.int32, L)

    def _dyn_gather(x, idx):
        return lax.gather(
            x, idx[:, None],
            dimension_numbers=lax.GatherDimensionNumbers(
                offset_dims=(), collapsed_slice_dims=(0,),
                start_index_map=(0,)),
            slice_sizes=(1,),
            mode=lax.GatherScatterMode.PROMISE_IN_BOUNDS)

    def _allsum(x):
        for sh in (1, 2, 4, 8):
            x = x + _dyn_gather(x, lane_iota ^ sh)
        return x

    def _proj_group(g, _):
        packed = jnp.zeros((L,), jnp.float32)
        for i in range(L):
            k = g * L + i
            acc = abuf[k, 0, pl.ds(0, L)] * wch[0]
            for c in range(1, D // L):
                acc = acc + abuf[k, 0, pl.ds(c * L, L)] * wch[c]
            packed = jnp.where(lane_iota == i, _allsum(acc), packed)
        pbuf[pl.ds(g * L, L)] = packed
        return _

    lax.fori_loop(0, APT // L, _proj_group, 0)
    pltpu.sync_copy(pbuf, proj_shared.at[pl.ds(base, APT)])
    plsc.subcore_barrier()
    pltpu.sync_copy(proj_shared, proj_v)

    bias_vec = bias_v[...]

    def _chunk(j, _):
        widx = wkr_v[pl.ds(j * L, L)]
        pv = plsc.load_gather(proj_v, [widx])
        s1 = 1.0 / (1.0 + jnp.exp(-(pv + bias_vec)))
        s2 = (1.0 - s1) * (1.0 / (R - 1))
        d = s1 - s2
        cols = [jnp.exp(rows_t[c, pl.ds(j * L, L)]) for c in range(R)]
        acc = cols
        while len(acc) > 1:
            acc = [a + b for a, b in zip(acc[::2], acc[1::2])]
        rinv = 1.0 / acc[0]
        for c in range(R):
            rows_t[c, pl.ds(j * L, L)] = s2 + d * (cols[c] * rinv)
        return _

    out_cps = []
    for g in range(TPW // 128):
        for cp in groups[g]:
            cp.wait()
        lax.fori_loop(g * 8, (g + 1) * 8, _chunk, 0)
        ocp = pltpu.make_async_copy(
            rows_t.at[:, pl.ds(g * 128, 128)],
            out_hbm.at[:, pl.ds(off + g * 128, 128)], osem)
        ocp.start()
        out_cps.append(ocp)
    for ocp in out_cps:
        ocp.wait()


def kernel(ability, labels, w_relation, bias, wkr, tsk):
    b16 = jnp.broadcast_to(bias.reshape(()), (L,))
    w1 = w_relation.reshape(D)
    ltf = jnp.transpose(labels.reshape(LBL, R)).reshape(R * LBL)
    out_t = _sc_score(ability, ltf, w1, b16, wkr, tsk)
    return jnp.transpose(out_t)[:, None, :]


# speedup vs baseline: 1.0387x; 1.0387x over previous
import functools

import jax
import jax.numpy as jnp
from jax import lax
from jax.experimental import pallas as pl
from jax.experimental.pallas import tpu as pltpu
from jax.experimental.pallas import tpu_sc as plsc

L = 16
NC = 2
NS = 16
NW = NC * NS

T = 16384
D = 128
R = 16
V = 1000
LBL = 100000

TPW = T // NW
APT = 64
ABASE_MAX = V - APT

_mesh = plsc.VectorSubcoreMesh(core_axis_name="c", subcore_axis_name="s")


@functools.partial(
    pl.kernel,
    out_type=jax.ShapeDtypeStruct((R, T), jnp.float32),
    mesh=_mesh,
    compiler_params=pltpu.CompilerParams(
        needs_layout_passes=False, use_tc_tiling_on_sc=False),
    scratch_types=[
        pltpu.VMEM((APT, 1, D), jnp.float32),
        pltpu.VMEM((D,), jnp.float32),
        pltpu.VMEM((L,), jnp.float32),
        pltpu.VMEM((TPW,), jnp.int32),
        pltpu.VMEM((TPW,), jnp.int32),
        pltpu.VMEM((R, TPW), jnp.float32),
        pltpu.VMEM((NS * APT,), jnp.float32),
        pltpu.VMEM((APT,), jnp.float32),
        pltpu.VMEM_SHARED((NS * APT,), jnp.float32),
        pltpu.SemaphoreType.DMA,
    ],
)
def _sc_score(ability_hbm, labels_hbm, w_hbm, bias_hbm, wkr_hbm, tsk_hbm,
              out_hbm, abuf, w_v, bias_v, wkr_v, tsk_v, rows_t, proj_v,
              pbuf, proj_shared, sem):
    cid = lax.axis_index("c")
    sid = lax.axis_index("s")
    wid = sid * NC + cid
    off = wid * TPW

    pltpu.sync_copy(wkr_hbm.at[pl.ds(off, TPW)], wkr_v)
    pltpu.sync_copy(tsk_hbm.at[pl.ds(off, TPW)], tsk_v)
    gathers = []
    for r in range(R):
        cp = pltpu.make_async_copy(
            labels_hbm.at[pl.ds(r * LBL, LBL)].at[tsk_v.at[pl.ds(0, TPW)]],
            rows_t.at[r, pl.ds(0, TPW)], sem)
        cp.start()
        gathers.append(cp)

    pltpu.sync_copy(w_hbm, w_v)
    pltpu.sync_copy(bias_hbm, bias_v)
    base = jnp.minimum(sid * APT, ABASE_MAX)
    pltpu.sync_copy(ability_hbm.at[pl.ds(base, APT)], abuf)
    wch = [w_v[pl.ds(c * L, L)] for c in range(D // L)]

    lane_iota = lax.iota(jnp.int32, L)

    def _dyn_gather(x, idx):
        return lax.gather(
            x, idx[:, None],
            dimension_numbers=lax.GatherDimensionNumbers(
                offset_dims=(), collapsed_slice_dims=(0,),
                start_index_map=(0,)),
            slice_sizes=(1,),
            mode=lax.GatherScatterMode.PROMISE_IN_BOUNDS)

    def _allsum(x):
        for sh in (1, 2, 4, 8):
            x = x + _dyn_gather(x, lane_iota ^ sh)
        return x

    def _proj_group(g, _):
        packed = jnp.zeros((L,), jnp.float32)
        for i in range(L):
            k = g * L + i
            acc = abuf[k, 0, pl.ds(0, L)] * wch[0]
            for c in range(1, D // L):
                acc = acc + abuf[k, 0, pl.ds(c * L, L)] * wch[c]
            packed = jnp.where(lane_iota == i, _allsum(acc), packed)
        pbuf[pl.ds(g * L, L)] = packed
        return _

    lax.fori_loop(0, APT // L, _proj_group, 0)
    pltpu.sync_copy(pbuf, proj_shared.at[pl.ds(base, APT)])
    plsc.subcore_barrier()
    pltpu.sync_copy(proj_shared, proj_v)

    bias_vec = bias_v[...]

    def _chunk(j, _):
        widx = wkr_v[pl.ds(j * L, L)]
        pv = plsc.load_gather(proj_v, [widx])
        s1 = 1.0 / (1.0 + jnp.exp(-(pv + bias_vec)))
        s2 = (1.0 - s1) * (1.0 / (R - 1))
        d = s1 - s2
        cols = [jnp.exp(rows_t[c, pl.ds(j * L, L)]) for c in range(R)]
        acc = cols
        while len(acc) > 1:
            acc = [a + b for a, b in zip(acc[::2], acc[1::2])]
        rinv = 1.0 / acc[0]
        for c in range(R):
            rows_t[c, pl.ds(j * L, L)] = s2 + d * (cols[c] * rinv)
        return _

    for cp in gathers:
        cp.wait()
    lax.fori_loop(0, TPW // L, _chunk, 0)
    pltpu.sync_copy(rows_t, out_hbm.at[:, pl.ds(off, TPW)])


def kernel(ability, labels, w_relation, bias, wkr, tsk):
    b16 = jnp.broadcast_to(bias.reshape(()), (L,))
    w1 = w_relation.reshape(D)
    ltf = jnp.transpose(labels.reshape(LBL, R)).reshape(R * LBL)
    out_t = _sc_score(ability, ltf, w1, b16, wkr, tsk)
    return jnp.transpose(out_t)[:, None, :]
